# baseline (speedup 1.0000x reference)
import functools

import jax
import jax.numpy as jnp
from jax import lax
from jax.experimental import pallas as pl
from jax.experimental.pallas import tpu as pltpu
from jax.experimental.pallas import tpu_sc as plsc

N = 10000
E = 320000
D = 128

NC = 2
NS = 16
NW = NC * NS
CHUNK = 128
STEPS = 80
PH = 2
PSTEPS = STEPS // PH
EPAD = NW * STEPS * CHUNK
NPAD = N + 8
RPS = 624

_mesh = plsc.VectorSubcoreMesh(
    core_axis_name="c", subcore_axis_name="s", num_cores=NC, num_subcores=NS
)


@functools.partial(
    pl.kernel,
    out_type=jax.ShapeDtypeStruct((NC, N, D), jnp.float32),
    mesh=_mesh,
    scratch_types=[
        pltpu.VMEM_SHARED((NPAD, D), jnp.float32),
        pltpu.VMEM((PSTEPS, CHUNK), jnp.int32),
        pltpu.VMEM((PSTEPS, CHUNK), jnp.int32),
        pltpu.VMEM((2, CHUNK, D), jnp.float32),
        pltpu.SemaphoreType.DMA,
    ],
)
def _sc_agg(h_hbm, src_hbm, dst_hbm, zer_hbm, out_hbm,
            agg_sh, src_v, dst_v, rows_v, gsem):
    c = lax.axis_index("c")
    s = lax.axis_index("s")
    wid = s * NC + c

    r0 = s * RPS

    @pl.when(jnp.logical_and(c == 0, s < NS - 1))
    def _():
        pltpu.sync_copy(h_hbm.at[pl.ds(r0, RPS)], agg_sh.at[pl.ds(r0, RPS)])

    @pl.when(jnp.logical_and(c == 0, s == NS - 1))
    def _():
        pltpu.sync_copy(h_hbm.at[pl.ds(9360, 640)], agg_sh.at[pl.ds(9360, 640)])

    @pl.when(jnp.logical_and(c != 0, s < NS - 1))
    def _():
        pltpu.sync_copy(zer_hbm.at[pl.ds(r0, RPS)], agg_sh.at[pl.ds(r0, RPS)])

    @pl.when(jnp.logical_and(c != 0, s == NS - 1))
    def _():
        pltpu.sync_copy(zer_hbm.at[pl.ds(9360, 640)], agg_sh.at[pl.ds(9360, 640)])

    plsc.subcore_barrier()

    def pair(k, carry):
        j0 = 2 * k
        pltpu.make_async_copy(h_hbm.at[src_v.at[j0]], rows_v.at[0], gsem).wait()
        pltpu.make_async_copy(
            h_hbm.at[src_v.at[j0 + 1]], rows_v.at[1], gsem
        ).start()
        pltpu.sync_copy(rows_v.at[0], agg_sh.at[dst_v.at[j0]], add=True)
        pltpu.make_async_copy(h_hbm.at[src_v.at[j0 + 1]], rows_v.at[1], gsem).wait()

        @pl.when(k < PSTEPS // 2 - 1)
        def _():
            pltpu.make_async_copy(
                h_hbm.at[src_v.at[j0 + 2]], rows_v.at[0], gsem
            ).start()

        pltpu.sync_copy(rows_v.at[1], agg_sh.at[dst_v.at[j0 + 1]], add=True)
        return carry

    for p in range(PH):
        pltpu.sync_copy(src_hbm.at[wid, pl.ds(p * PSTEPS, PSTEPS)], src_v)
        pltpu.sync_copy(dst_hbm.at[wid, pl.ds(p * PSTEPS, PSTEPS)], dst_v)
        pltpu.make_async_copy(h_hbm.at[src_v.at[0]], rows_v.at[0], gsem).start()
        lax.fori_loop(0, PSTEPS // 2, pair, 0)

    plsc.subcore_barrier()

    @pl.when(s < NS - 1)
    def _():
        pltpu.sync_copy(agg_sh.at[pl.ds(r0, RPS)], out_hbm.at[c, pl.ds(r0, RPS)])

    @pl.when(s == NS - 1)
    def _():
        pltpu.sync_copy(agg_sh.at[pl.ds(9360, 640)], out_hbm.at[c, pl.ds(9360, 640)])


def _mlp_body(p_ref, w_ref, b_ref, a_ref, o_ref):
    h = p_ref[0] + p_ref[1]
    z = jnp.dot(h, w_ref[:], preferred_element_type=jnp.float32) + b_ref[:]
    o_ref[:] = jnp.where(z >= 0.0, z, a_ref[:] * z)


_MB = 1000


def _tc_mlp(p, W, b2d, a2d):
    return pl.pallas_call(
        _mlp_body,
        grid=(N // _MB,),
        in_specs=[
            pl.BlockSpec((2, _MB, D), lambda i: (0, i, 0)),
            pl.BlockSpec((D, D), lambda i: (0, 0)),
            pl.BlockSpec((1, D), lambda i: (0, 0)),
            pl.BlockSpec((1, D), lambda i: (0, 0)),
        ],
        out_specs=pl.BlockSpec((_MB, D), lambda i: (i, 0)),
        out_shape=jax.ShapeDtypeStruct((N, D), jnp.float32),
    )(p, W, b2d, a2d)


def kernel(x, edge_index, W1, b1, a1, W2, b2, a2):
    src = edge_index[0]
    dst = edge_index[1]
    pad = EPAD - E
    src_p = jnp.concatenate([src, jnp.zeros((pad,), jnp.int32)])
    dst_p = jnp.concatenate([dst, jnp.full((pad,), N, jnp.int32)])
    src_p = src_p.reshape(NW, STEPS, CHUNK)
    dst_p = dst_p.reshape(NW, STEPS, CHUNK)
    zer = jnp.zeros((N, D), jnp.float32)
    b1r = b1.reshape(1, D)
    b2r = b2.reshape(1, D)
    a1r = jnp.full((1, D), a1, jnp.float32)
    a2r = jnp.full((1, D), a2, jnp.float32)

    p1 = _sc_agg(x, src_p, dst_p, zer)
    h1 = _tc_mlp(p1, W1, b1r, a1r)
    p2 = _sc_agg(h1, src_p, dst_p, zer)
    h2 = _tc_mlp(p2, W2, b2r, a2r)
    return h2


# speedup vs baseline: 1.0205x; 1.0205x over previous
import functools

import jax
import jax.numpy as jnp
from jax import lax
from jax.experimental import pallas as pl
from jax.experimental.pallas import tpu as pltpu
from jax.experimental.pallas import tpu_sc as plsc

N = 10000
E = 320000
D = 128

NC = 2
NS = 16
NW = NC * NS
CHUNK = 128
STEPS = 80
PH = 2
PSTEPS = STEPS // PH
EPAD = NW * STEPS * CHUNK
NPAD = N + 8
RPS = 624

_mesh = plsc.VectorSubcoreMesh(
    core_axis_name="c", subcore_axis_name="s", num_cores=NC, num_subcores=NS
)


@functools.partial(
    pl.kernel,
    out_type=jax.ShapeDtypeStruct((NC, N, D), jnp.float32),
    mesh=_mesh,
    scratch_types=[
        pltpu.VMEM_SHARED((NPAD, D), jnp.float32),
        pltpu.VMEM((PSTEPS, CHUNK), jnp.int32),
        pltpu.VMEM((PSTEPS, CHUNK), jnp.int32),
        pltpu.VMEM((2, CHUNK, D), jnp.float32),
        pltpu.SemaphoreType.DMA,
        pltpu.SemaphoreType.DMA,
    ],
)
def _sc_agg(h_hbm, src_hbm, dst_hbm, zer_hbm, out_hbm,
            agg_sh, src_v, dst_v, rows_v, gsem, ssem):
    c = lax.axis_index("c")
    s = lax.axis_index("s")
    wid = s * NC + c

    r0 = s * RPS

    @pl.when(jnp.logical_and(c == 0, s < NS - 1))
    def _():
        pltpu.sync_copy(h_hbm.at[pl.ds(r0, RPS)], agg_sh.at[pl.ds(r0, RPS)])

    @pl.when(jnp.logical_and(c == 0, s == NS - 1))
    def _():
        pltpu.sync_copy(h_hbm.at[pl.ds(9360, 640)], agg_sh.at[pl.ds(9360, 640)])

    @pl.when(jnp.logical_and(c != 0, s < NS - 1))
    def _():
        pltpu.sync_copy(zer_hbm.at[pl.ds(r0, RPS)], agg_sh.at[pl.ds(r0, RPS)])

    @pl.when(jnp.logical_and(c != 0, s == NS - 1))
    def _():
        pltpu.sync_copy(zer_hbm.at[pl.ds(9360, 640)], agg_sh.at[pl.ds(9360, 640)])

    plsc.subcore_barrier()

    def gstart(j, b):
        pltpu.make_async_copy(h_hbm.at[src_v.at[j]], rows_v.at[b], gsem).start()

    def gwait():
        pltpu.make_async_copy(h_hbm.at[src_v.at[0]], rows_v.at[0], gsem).wait()

    def sstart(j, b):
        pltpu.async_copy(rows_v.at[b], agg_sh.at[dst_v.at[j]], ssem, add=True)

    def swait():
        pltpu.make_async_copy(rows_v.at[0], agg_sh.at[dst_v.at[0]], ssem).wait()

    nhalf = PSTEPS // 2

    def pair(k, carry):
        j0 = 2 * k
        gwait()
        sstart(j0, 0)
        gwait()
        sstart(j0 + 1, 1)
        swait()

        @pl.when(k < nhalf - 1)
        def _():
            gstart(j0 + 2, 0)

        swait()

        @pl.when(k < nhalf - 1)
        def _():
            gstart(j0 + 3, 1)

        return carry

    for p in range(PH):
        pltpu.sync_copy(src_hbm.at[wid, pl.ds(p * PSTEPS, PSTEPS)], src_v)
        pltpu.sync_copy(dst_hbm.at[wid, pl.ds(p * PSTEPS, PSTEPS)], dst_v)
        gstart(0, 0)
        gstart(1, 1)
        lax.fori_loop(0, nhalf, pair, 0)

    plsc.subcore_barrier()

    @pl.when(s < NS - 1)
    def _():
        pltpu.sync_copy(agg_sh.at[pl.ds(r0, RPS)], out_hbm.at[c, pl.ds(r0, RPS)])

    @pl.when(s == NS - 1)
    def _():
        pltpu.sync_copy(agg_sh.at[pl.ds(9360, 640)], out_hbm.at[c, pl.ds(9360, 640)])


def _mlp_body(p_ref, w_ref, b_ref, a_ref, o_ref):
    h = p_ref[0] + p_ref[1]
    z = jnp.dot(h, w_ref[:], preferred_element_type=jnp.float32) + b_ref[:]
    o_ref[:] = jnp.where(z >= 0.0, z, a_ref[:] * z)


_MB = 1000


def _tc_mlp(p, W, b2d, a2d):
    return pl.pallas_call(
        _mlp_body,
        grid=(N // _MB,),
        in_specs=[
            pl.BlockSpec((2, _MB, D), lambda i: (0, i, 0)),
            pl.BlockSpec((D, D), lambda i: (0, 0)),
            pl.BlockSpec((1, D), lambda i: (0, 0)),
            pl.BlockSpec((1, D), lambda i: (0, 0)),
        ],
        out_specs=pl.BlockSpec((_MB, D), lambda i: (i, 0)),
        out_shape=jax.ShapeDtypeStruct((N, D), jnp.float32),
    )(p, W, b2d, a2d)


def kernel(x, edge_index, W1, b1, a1, W2, b2, a2):
    src = edge_index[0]
    dst = edge_index[1]
    pad = EPAD - E
    src_p = jnp.concatenate([src, jnp.zeros((pad,), jnp.int32)])
    dst_p = jnp.concatenate([dst, jnp.full((pad,), N, jnp.int32)])
    src_p = src_p.reshape(NW, STEPS, CHUNK)
    dst_p = dst_p.reshape(NW, STEPS, CHUNK)
    zer = jnp.zeros((N, D), jnp.float32)
    b1r = b1.reshape(1, D)
    b2r = b2.reshape(1, D)
    a1r = jnp.full((1, D), a1, jnp.float32)
    a2r = jnp.full((1, D), a2, jnp.float32)

    p1 = _sc_agg(x, src_p, dst_p, zer)
    h1 = _tc_mlp(p1, W1, b1r, a1r)
    p2 = _sc_agg(h1, src_p, dst_p, zer)
    h2 = _tc_mlp(p2, W2, b2r, a2r)
    return h2


# speedup vs baseline: 1.5775x; 1.5458x over previous
import functools

import jax
import jax.numpy as jnp
from jax import lax
from jax.experimental import pallas as pl
from jax.experimental.pallas import tpu as pltpu
from jax.experimental.pallas import tpu_sc as plsc

N = 10000
E = 320000
D = 128

NC = 2
NS = 16
NW = NC * NS
CHUNK = 128
STEPS = 79
EPW = E // NW
PADW = STEPS * CHUNK - EPW
NPAD = N + 8
RPS = 624

_mesh = plsc.VectorSubcoreMesh(
    core_axis_name="c", subcore_axis_name="s", num_cores=NC, num_subcores=NS
)


@functools.partial(
    pl.kernel,
    out_type=jax.ShapeDtypeStruct((NC, N, D), jnp.float32),
    mesh=_mesh,
    scratch_types=[
        pltpu.VMEM_SHARED((NPAD, D), jnp.float32),
        pltpu.VMEM((STEPS, CHUNK), jnp.int32),
        pltpu.VMEM((STEPS, CHUNK), jnp.int32),
        pltpu.VMEM((CHUNK, D), jnp.float32),
        pltpu.SemaphoreType.DMA,
    ],
)
def _sc_agg(h_hbm, src_hbm, dst_hbm, zer_hbm, out_hbm,
            agg_sh, src_v, dst_v, rows_v, gsem):
    c = lax.axis_index("c")
    s = lax.axis_index("s")
    wid = s * NC + c

    pltpu.sync_copy(src_hbm.at[wid], src_v)
    pltpu.sync_copy(dst_hbm.at[wid], dst_v)

    r0 = s * RPS

    @pl.when(jnp.logical_and(c == 0, s < NS - 1))
    def _():
        pltpu.sync_copy(h_hbm.at[pl.ds(r0, RPS)], agg_sh.at[pl.ds(r0, RPS)])

    @pl.when(jnp.logical_and(c == 0, s == NS - 1))
    def _():
        pltpu.sync_copy(h_hbm.at[pl.ds(9360, 640)], agg_sh.at[pl.ds(9360, 640)])
        pltpu.sync_copy(zer_hbm.at[pl.ds(N, 8)], agg_sh.at[pl.ds(N, 8)])

    @pl.when(jnp.logical_and(c != 0, s < NS - 1))
    def _():
        pltpu.sync_copy(zer_hbm.at[pl.ds(r0, RPS)], agg_sh.at[pl.ds(r0, RPS)])

    @pl.when(jnp.logical_and(c != 0, s == NS - 1))
    def _():
        pltpu.sync_copy(zer_hbm.at[pl.ds(9360, 648)], agg_sh.at[pl.ds(9360, 648)])

    plsc.subcore_barrier()

    def step(j, carry):
        pltpu.async_copy(h_hbm.at[src_v.at[j]], rows_v, gsem).wait()
        pltpu.sync_copy(rows_v, agg_sh.at[dst_v.at[j]], add=True)
        return carry

    lax.fori_loop(0, STEPS, step, 0)

    plsc.subcore_barrier()

    @pl.when(s < NS - 1)
    def _():
        pltpu.sync_copy(agg_sh.at[pl.ds(r0, RPS)], out_hbm.at[c, pl.ds(r0, RPS)])

    @pl.when(s == NS - 1)
    def _():
        pltpu.sync_copy(agg_sh.at[pl.ds(9360, 640)], out_hbm.at[c, pl.ds(9360, 640)])


def _mlp_body(p_ref, w_ref, b_ref, a_ref, o_ref):
    h = p_ref[0] + p_ref[1]
    z = jnp.dot(h, w_ref[:], preferred_element_type=jnp.float32) + b_ref[:]
    o_ref[:] = jnp.where(z >= 0.0, z, a_ref[:] * z)


_MB = 1000


def _tc_mlp(p, W, b2d, a2d):
    return pl.pallas_call(
        _mlp_body,
        grid=(N // _MB,),
        in_specs=[
            pl.BlockSpec((2, _MB, D), lambda i: (0, i, 0)),
            pl.BlockSpec((D, D), lambda i: (0, 0)),
            pl.BlockSpec((1, D), lambda i: (0, 0)),
            pl.BlockSpec((1, D), lambda i: (0, 0)),
        ],
        out_specs=pl.BlockSpec((_MB, D), lambda i: (i, 0)),
        out_shape=jax.ShapeDtypeStruct((N, D), jnp.float32),
    )(p, W, b2d, a2d)


def kernel(x, edge_index, W1, b1, a1, W2, b2, a2):
    src = edge_index[0].reshape(NW, EPW)
    dst = edge_index[1].reshape(NW, EPW)
    src_pad = jnp.zeros((NW, PADW), jnp.int32)
    dst_pad = jnp.broadcast_to(
        N + (jnp.arange(PADW, dtype=jnp.int32) % 8), (NW, PADW)
    )
    src_p = jnp.concatenate([src, src_pad], axis=1).reshape(NW, STEPS, CHUNK)
    dst_p = jnp.concatenate([dst, dst_pad], axis=1).reshape(NW, STEPS, CHUNK)
    zer = jnp.zeros((NPAD, D), jnp.float32)
    b1r = b1.reshape(1, D)
    b2r = b2.reshape(1, D)
    a1r = jnp.full((1, D), a1, jnp.float32)
    a2r = jnp.full((1, D), a2, jnp.float32)

    p1 = _sc_agg(x, src_p, dst_p, zer)
    h1 = _tc_mlp(p1, W1, b1r, a1r)
    p2 = _sc_agg(h1, src_p, dst_p, zer)
    h2 = _tc_mlp(p2, W2, b2r, a2r)
    return h2
